# initial kernel scaffold (unmeasured)
import jax
import jax.numpy as jnp
from jax import lax
from jax.experimental import pallas as pl
from jax.experimental.pallas import tpu as pltpu

P = 8


def kernel(x, w_mat):
    m_per, k = x.shape
    n = w_mat.shape[1]
    n_per = n // P
    m_tot = m_per * P

    def body(x_ref, w_ref, out_ref, send_buf, recv_buf, send_sems, recv_sems):
        my = lax.axis_index("i")

        bar = pltpu.get_barrier_semaphore()
        for d in range(P):
            @pl.when(d != my)
            def _():
                pl.semaphore_signal(
                    bar, inc=1,
                    device_id=(d,), device_id_type=pl.DeviceIdType.MESH,
                )
        pl.semaphore_wait(bar, P - 1)

        sends = []
        for j in range(P):
            blk = jnp.dot(
                x_ref[:, :], w_ref[:, j * n_per:(j + 1) * n_per],
                preferred_element_type=jnp.float32,
            )
            blk = jnp.maximum(blk, 0.0)

            @pl.when(j == my)
            def _():
                out_ref[j * m_per:(j + 1) * m_per, :] = blk

            @pl.when(j != my)
            def _():
                send_buf[j, :, :] = blk.astype(jnp.bfloat16)

            rdma = pltpu.make_async_remote_copy(
                src_ref=send_buf.at[j],
                dst_ref=recv_buf.at[my],
                send_sem=send_sems.at[j],
                recv_sem=recv_sems.at[my],
                device_id=(j,),
                device_id_type=pl.DeviceIdType.MESH,
            )

            @pl.when(j != my)
            def _():
                rdma.start()

            sends.append((j, rdma))

        for src in range(P):
            recv = pltpu.make_async_remote_copy(
                src_ref=send_buf.at[src],
                dst_ref=recv_buf.at[src],
                send_sem=send_sems.at[src],
                recv_sem=recv_sems.at[src],
                device_id=(src,),
                device_id_type=pl.DeviceIdType.MESH,
            )

            @pl.when(src != my)
            def _():
                recv.wait_recv()
                out_ref[src * m_per:(src + 1) * m_per, :] = (
                    recv_buf[src, :, :].astype(jnp.float32)
                )

        for j, rdma in sends:
            @pl.when(j != my)
            def _():
                rdma.wait_send()

    return pl.pallas_call(
        body,
        out_shape=jax.ShapeDtypeStruct((m_tot, n_per), jnp.float32),
        in_specs=[
            pl.BlockSpec(memory_space=pltpu.VMEM),
            pl.BlockSpec(memory_space=pltpu.VMEM),
        ],
        out_specs=pl.BlockSpec(memory_space=pltpu.VMEM),
        scratch_shapes=[
            pltpu.VMEM((P, m_per, n_per), jnp.bfloat16),
            pltpu.VMEM((P, m_per, n_per), jnp.bfloat16),
            pltpu.SemaphoreType.DMA((P,)),
            pltpu.SemaphoreType.DMA((P,)),
        ],
        compiler_params=pltpu.CompilerParams(collective_id=0),
    )(x, w_mat)


# baseline (device time: 223505 ns/iter reference)
import jax
import jax.numpy as jnp
from jax import lax
from jax.experimental import pallas as pl
from jax.experimental.pallas import tpu as pltpu

P = 8


def kernel(x, w_mat):
    m_per, k = x.shape
    n = w_mat.shape[1]
    n_per = n // P
    m_tot = m_per * P

    x = x.astype(jnp.bfloat16)
    w_mat = w_mat.astype(jnp.bfloat16)

    def body(x_ref, w_ref, out_ref, send_buf, send_sems, recv_sems):
        my = lax.axis_index("i")
        j = pl.program_id(0)

        bar = pltpu.get_barrier_semaphore()

        @pl.when(j == 0)
        def _():
            for d in range(P):
                @pl.when(d != my)
                def _():
                    pl.semaphore_signal(
                        bar, inc=1,
                        device_id=(d,), device_id_type=pl.DeviceIdType.MESH,
                    )
            pl.semaphore_wait(bar, P - 1)

        part = jnp.dot(
            x_ref[:, :], w_ref[:, :], preferred_element_type=jnp.float32,
        )
        part = jnp.maximum(part, 0.0).astype(jnp.bfloat16)

        @pl.when(j == my)
        def _():
            out_ref[pl.ds(my * m_per, m_per), :] = part

        @pl.when(j != my)
        def _():
            send_buf[j, :, :] = part
            rdma = pltpu.make_async_remote_copy(
                src_ref=send_buf.at[j],
                dst_ref=out_ref.at[pl.ds(my * m_per, m_per), :],
                send_sem=send_sems.at[j],
                recv_sem=recv_sems.at[my],
                device_id=(j,),
                device_id_type=pl.DeviceIdType.MESH,
            )
            rdma.start()

        @pl.when(j == P - 1)
        def _():
            for src in range(P):
                recv = pltpu.make_async_remote_copy(
                    src_ref=send_buf.at[src],
                    dst_ref=out_ref.at[src * m_per:(src + 1) * m_per, :],
                    send_sem=send_sems.at[src],
                    recv_sem=recv_sems.at[src],
                    device_id=(src,),
                    device_id_type=pl.DeviceIdType.MESH,
                )

                @pl.when(src != my)
                def _():
                    recv.wait_recv()

            for d in range(P):
                send = pltpu.make_async_remote_copy(
                    src_ref=send_buf.at[d],
                    dst_ref=out_ref.at[d * m_per:(d + 1) * m_per, :],
                    send_sem=send_sems.at[d],
                    recv_sem=recv_sems.at[d],
                    device_id=(d,),
                    device_id_type=pl.DeviceIdType.MESH,
                )

                @pl.when(d != my)
                def _():
                    send.wait_send()

    return pl.pallas_call(
        body,
        grid=(P,),
        out_shape=jax.ShapeDtypeStruct((m_tot, n_per), jnp.bfloat16),
        in_specs=[
            pl.BlockSpec((m_per, k), lambda j: (0, 0)),
            pl.BlockSpec((k, n_per), lambda j: (0, j)),
        ],
        out_specs=pl.BlockSpec((m_tot, n_per), lambda j: (0, 0)),
        scratch_shapes=[
            pltpu.VMEM((P, m_per, n_per), jnp.bfloat16),
            pltpu.SemaphoreType.DMA((P,)),
            pltpu.SemaphoreType.DMA((P,)),
        ],
        compiler_params=pltpu.CompilerParams(
            collective_id=0,
            vmem_limit_bytes=52 * 1024 * 1024,
            dimension_semantics=("arbitrary",),
        ),
    )(x, w_mat)


# device time: 118830 ns/iter; 1.8809x vs baseline; 1.8809x over previous
import jax
import jax.numpy as jnp
from jax import lax
from jax.experimental import pallas as pl
from jax.experimental.pallas import tpu as pltpu

P = 8
CPB = 2
NSTEP = P * CPB


def kernel(x, w_mat):
    m_per, k = x.shape
    n = w_mat.shape[1]
    n_per = n // P
    nc = n_per // CPB
    m_tot = m_per * P

    x = x.astype(jnp.bfloat16)

    def body(x_ref, w_ref, out_ref, w_stage, send_buf,
             w_sems, loc_sems, send_sems, recv_sems):
        my = lax.axis_index("i")
        idx = pl.program_id(0)

        def dest(i):
            return lax.rem(my + 1 + lax.div(i, CPB), P)

        def chunk(i):
            return lax.rem(i, CPB)

        def w_fetch(i, slot):
            col = dest(i) * n_per + chunk(i) * nc
            return pltpu.make_async_copy(
                w_ref.at[:, pl.ds(col, nc)], w_stage.at[slot], w_sems.at[slot],
            )

        bar = pltpu.get_barrier_semaphore()

        @pl.when(idx == 0)
        def _():
            w_fetch(idx, 0).start()
            for d in range(P):
                @pl.when(d != my)
                def _():
                    pl.semaphore_signal(
                        bar, inc=1,
                        device_id=(d,), device_id_type=pl.DeviceIdType.MESH,
                    )
            pl.semaphore_wait(bar, P - 1)

        slot = lax.rem(idx, 2)

        @pl.when(idx < NSTEP - 1)
        def _():
            w_fetch(idx + 1, 1 - slot).start()

        w_fetch(idx, slot).wait()

        part = jnp.dot(
            x_ref[:, :], w_stage[slot].astype(jnp.bfloat16),
            preferred_element_type=jnp.float32,
        )
        part = jnp.maximum(part, 0.0).astype(jnp.bfloat16)
        send_buf[idx, :, :] = part

        jj = dest(idx)
        cc = chunk(idx)

        @pl.when(jj != my)
        def _():
            rdma = pltpu.make_async_remote_copy(
                src_ref=send_buf.at[idx],
                dst_ref=out_ref.at[pl.ds(my * m_per, m_per), pl.ds(cc * nc, nc)],
                send_sem=send_sems.at[idx],
                recv_sem=recv_sems.at[my * CPB + cc],
                device_id=(jj,),
                device_id_type=pl.DeviceIdType.MESH,
            )
            rdma.start()

        @pl.when(idx == NSTEP - 1)
        def _():
            for c in range(CPB):
                pltpu.make_async_copy(
                    send_buf.at[NSTEP - CPB + c],
                    out_ref.at[pl.ds(my * m_per, m_per), c * nc:(c + 1) * nc],
                    loc_sems.at[c],
                ).start()

            for src in range(P):
                for c in range(CPB):
                    recv = pltpu.make_async_remote_copy(
                        src_ref=send_buf.at[src * CPB + c],
                        dst_ref=out_ref.at[
                            src * m_per:(src + 1) * m_per, c * nc:(c + 1) * nc
                        ],
                        send_sem=send_sems.at[src * CPB + c],
                        recv_sem=recv_sems.at[src * CPB + c],
                        device_id=(src,),
                        device_id_type=pl.DeviceIdType.MESH,
                    )

                    @pl.when(src != my)
                    def _():
                        recv.wait_recv()

            for i in range(NSTEP - CPB):
                send = pltpu.make_async_remote_copy(
                    src_ref=send_buf.at[i],
                    dst_ref=out_ref.at[0:m_per, 0:nc],
                    send_sem=send_sems.at[i],
                    recv_sem=recv_sems.at[0],
                    device_id=(0,),
                    device_id_type=pl.DeviceIdType.MESH,
                )
                send.wait_send()

            for c in range(CPB):
                pltpu.make_async_copy(
                    send_buf.at[NSTEP - CPB + c],
                    out_ref.at[pl.ds(my * m_per, m_per), c * nc:(c + 1) * nc],
                    loc_sems.at[c],
                ).wait()

    return pl.pallas_call(
        body,
        grid=(NSTEP,),
        out_shape=jax.ShapeDtypeStruct((m_tot, n_per), jnp.bfloat16),
        in_specs=[
            pl.BlockSpec((m_per, k), lambda i: (0, 0)),
            pl.BlockSpec(memory_space=pl.ANY),
        ],
        out_specs=pl.BlockSpec((m_tot, n_per), lambda i: (0, 0)),
        scratch_shapes=[
            pltpu.VMEM((2, k, nc), jnp.float32),
            pltpu.VMEM((NSTEP, m_per, nc), jnp.bfloat16),
            pltpu.SemaphoreType.DMA((2,)),
            pltpu.SemaphoreType.DMA((CPB,)),
            pltpu.SemaphoreType.DMA((NSTEP,)),
            pltpu.SemaphoreType.DMA((NSTEP,)),
        ],
        compiler_params=pltpu.CompilerParams(
            collective_id=0,
            vmem_limit_bytes=56 * 1024 * 1024,
            dimension_semantics=("arbitrary",),
        ),
    )(x, w_mat)


# device time: 118684 ns/iter; 1.8832x vs baseline; 1.0012x over previous
import jax
import jax.numpy as jnp
from jax import lax
from jax.experimental import pallas as pl
from jax.experimental.pallas import tpu as pltpu

P = 8
CPB = 2
NSTEP = P * CPB


def kernel(x, w_mat):
    m_per, k = x.shape
    n = w_mat.shape[1]
    n_per = n // P
    nc = n_per // CPB
    m_tot = m_per * P

    x = x.astype(jnp.bfloat16)

    def body(x_ref, w_ref, out_ref, w_stage, send_buf,
             w_sems, loc_sems, send_sems, recv_sems):
        my = lax.axis_index("i")
        idx = pl.program_id(0)

        def dest(i):
            return lax.rem(my + 1 + lax.div(i, CPB), P)

        def chunk(i):
            return lax.rem(i, CPB)

        def w_fetch(i, slot):
            col = dest(i) * n_per + chunk(i) * nc
            return pltpu.make_async_copy(
                w_ref.at[:, pl.ds(col, nc)], w_stage.at[slot], w_sems.at[slot],
            )

        bar = pltpu.get_barrier_semaphore()

        @pl.when(idx == 0)
        def _():
            w_fetch(idx, 0).start()
            for d in range(P):
                @pl.when(d != my)
                def _():
                    pl.semaphore_signal(
                        bar, inc=1,
                        device_id=(d,), device_id_type=pl.DeviceIdType.MESH,
                    )
            pl.semaphore_wait(bar, P - 1)

        slot = lax.rem(idx, 2)

        @pl.when(idx < NSTEP - 1)
        def _():
            w_fetch(idx + 1, 1 - slot).start()

        w_fetch(idx, slot).wait()

        part = jnp.dot(
            x_ref[:, :], w_stage[slot].astype(jnp.bfloat16),
            preferred_element_type=jnp.float32,
        )
        part = jnp.maximum(part, 0.0).astype(jnp.bfloat16)
        send_buf[idx, :, :] = part

        jj = dest(idx)
        cc = chunk(idx)

        @pl.when(jj != my)
        def _():
            rdma = pltpu.make_async_remote_copy(
                src_ref=send_buf.at[idx],
                dst_ref=out_ref.at[pl.ds(my * m_per, m_per), pl.ds(cc * nc, nc)],
                send_sem=send_sems.at[idx],
                recv_sem=recv_sems.at[my * CPB + cc],
                device_id=(jj,),
                device_id_type=pl.DeviceIdType.MESH,
            )
            rdma.start()

        @pl.when(idx == NSTEP - 1)
        def _():
            for c in range(CPB):
                pltpu.make_async_copy(
                    send_buf.at[NSTEP - CPB + c],
                    out_ref.at[pl.ds(my * m_per, m_per), c * nc:(c + 1) * nc],
                    loc_sems.at[c],
                ).start()

            for src in range(P):
                for c in range(CPB):
                    recv = pltpu.make_async_remote_copy(
                        src_ref=send_buf.at[src * CPB + c],
                        dst_ref=out_ref.at[
                            src * m_per:(src + 1) * m_per, c * nc:(c + 1) * nc
                        ],
                        send_sem=send_sems.at[src * CPB + c],
                        recv_sem=recv_sems.at[src * CPB + c],
                        device_id=(src,),
                        device_id_type=pl.DeviceIdType.MESH,
                    )

                    @pl.when(src != my)
                    def _():
                        recv.wait_recv()

            for i in range(NSTEP - CPB):
                send = pltpu.make_async_remote_copy(
                    src_ref=send_buf.at[i],
                    dst_ref=out_ref.at[0:m_per, 0:nc],
                    send_sem=send_sems.at[i],
                    recv_sem=recv_sems.at[0],
                    device_id=(0,),
                    device_id_type=pl.DeviceIdType.MESH,
                )
                send.wait_send()

            for c in range(CPB):
                pltpu.make_async_copy(
                    send_buf.at[NSTEP - CPB + c],
                    out_ref.at[pl.ds(my * m_per, m_per), c * nc:(c + 1) * nc],
                    loc_sems.at[c],
                ).wait()

    return pl.pallas_call(
        body,
        grid=(NSTEP,),
        out_shape=jax.ShapeDtypeStruct((m_tot, n_per), jnp.bfloat16),
        in_specs=[
            pl.BlockSpec(memory_space=pltpu.VMEM),
            pl.BlockSpec(memory_space=pl.ANY),
        ],
        out_specs=pl.BlockSpec(memory_space=pltpu.VMEM),
        scratch_shapes=[
            pltpu.VMEM((2, k, nc), jnp.float32),
            pltpu.VMEM((NSTEP, m_per, nc), jnp.bfloat16),
            pltpu.SemaphoreType.DMA((2,)),
            pltpu.SemaphoreType.DMA((CPB,)),
            pltpu.SemaphoreType.DMA((NSTEP,)),
            pltpu.SemaphoreType.DMA((NSTEP,)),
        ],
        compiler_params=pltpu.CompilerParams(
            collective_id=0,
            vmem_limit_bytes=56 * 1024 * 1024,
            dimension_semantics=("arbitrary",),
        ),
    )(x, w_mat)
